# baseline (speedup 1.0000x reference)
import jax
import jax.numpy as jnp
from jax import lax
from jax.experimental import pallas as pl
from jax.experimental.pallas import tpu as pltpu
from jax.experimental.pallas import tpu_sc as plsc

_CH = 64
_NCH = 1024 // _CH


def _sc_body(x_hbm, kk_hbm, tok_hbm, out_hbm, kk_v, mask_buf, sem):
    b = lax.axis_index("c")
    t = lax.axis_index("s")

    pltpu.sync_copy(kk_hbm, kk_v.at[pl.ds(0, 16)])
    kk_t = kk_v[pl.ds(t, 16)][0]

    for i in range(_CH):
        pltpu.async_copy(tok_hbm, mask_buf.at[i], sem)
    pltpu.make_async_copy(x_hbm.at[b, t, pl.ds(0, _CH)], mask_buf, sem).wait()

    for c in range(_NCH):
        lo = c * _CH
        hi = lo + _CH
        sl = pl.ds(lo, _CH)

        @pl.when(kk_t >= hi)
        def _copy():
            pltpu.async_copy(x_hbm.at[b, t, sl], out_hbm.at[b, t, sl], sem)

        @pl.when(kk_t <= lo)
        def _fill():
            pltpu.async_copy(mask_buf, out_hbm.at[b, t, sl], sem)

        @pl.when(jnp.logical_and(kk_t > lo, kk_t < hi))
        def _mixed():
            def copy_row(i, carry):
                pltpu.async_copy(x_hbm.at[b, t, lo + i],
                                 out_hbm.at[b, t, lo + i], sem)
                return carry

            lax.fori_loop(0, kk_t - lo, copy_row, 0)

            def fill_row(i, carry):
                pltpu.async_copy(mask_buf.at[0],
                                 out_hbm.at[b, t, kk_t + i], sem)
                return carry

            lax.fori_loop(0, hi - kk_t, fill_row, 0)

    pltpu.make_async_copy(x_hbm.at[b, t], out_hbm.at[b, t], sem).wait()


def kernel(x, keep_k, mask_token):
    mesh = plsc.VectorSubcoreMesh(core_axis_name="c", subcore_axis_name="s")
    f = pl.kernel(
        _sc_body,
        out_type=jax.ShapeDtypeStruct(x.shape, x.dtype),
        mesh=mesh,
        scratch_types=[
            pltpu.VMEM((32,), jnp.int32),
            pltpu.VMEM((_CH, x.shape[-1]), jnp.float32),
            pltpu.SemaphoreType.DMA,
        ],
    )
    return f(x, keep_k.astype(jnp.int32), mask_token)


# speedup vs baseline: 8.6637x; 8.6637x over previous
import jax
import jax.numpy as jnp
from jax import lax
from jax.experimental import pallas as pl
from jax.experimental.pallas import tpu as pltpu
from jax.experimental.pallas import tpu_sc as plsc

_CH = 64
_NCH = 1024 // _CH


def _sc_body(x_hbm, kk_hbm, tok_hbm, out_hbm, kk_v, mask_buf, buf, sem):
    b = lax.axis_index("c")
    t = lax.axis_index("s")

    pltpu.sync_copy(kk_hbm, kk_v.at[pl.ds(0, 16)])
    kk_t = kk_v[pl.ds(t, 16)][0]

    for i in range(_CH):
        pltpu.async_copy(tok_hbm, mask_buf.at[i], sem)
    pltpu.make_async_copy(x_hbm.at[b, t, pl.ds(0, _CH)], mask_buf, sem).wait()

    q = kk_t // _CH
    r = kk_t % _CH
    qf = q + jnp.where(r > 0, 1, 0).astype(q.dtype)

    def fill_chunk(j, carry):
        pltpu.async_copy(mask_buf, out_hbm.at[b, t, pl.ds(j * _CH, _CH)], sem)
        return carry

    lax.fori_loop(qf, _NCH, fill_chunk, 0)

    def copy_chunk(i, carry):
        sl = pl.ds(i * _CH, _CH)
        pltpu.sync_copy(x_hbm.at[b, t, sl], buf)
        pltpu.sync_copy(buf, out_hbm.at[b, t, sl])
        return carry

    lax.fori_loop(0, q, copy_chunk, 0)

    @pl.when(r > 0)
    def _mixed():
        sl = pl.ds(q * _CH, _CH)
        pltpu.sync_copy(x_hbm.at[b, t, sl], buf)

        def row(n, carry):
            @pl.when(n < r)
            def _():
                pltpu.async_copy(buf.at[n], out_hbm.at[b, t, q * _CH + n], sem)

            @pl.when(n >= r)
            def _():
                pltpu.async_copy(mask_buf.at[0],
                                 out_hbm.at[b, t, q * _CH + n], sem)

            return carry

        lax.fori_loop(0, _CH, row, 0)

    def drain(j, carry):
        pltpu.make_async_copy(x_hbm.at[b, t, pl.ds(0, _CH)],
                              out_hbm.at[b, t, pl.ds(j * _CH, _CH)], sem).wait()
        return carry

    lax.fori_loop(qf, _NCH, drain, 0)

    @pl.when(r > 0)
    def _drain_mixed():
        pltpu.make_async_copy(x_hbm.at[b, t, pl.ds(0, _CH)],
                              out_hbm.at[b, t, pl.ds(q * _CH, _CH)], sem).wait()


def kernel(x, keep_k, mask_token):
    mesh = plsc.VectorSubcoreMesh(core_axis_name="c", subcore_axis_name="s")
    f = pl.kernel(
        _sc_body,
        out_type=jax.ShapeDtypeStruct(x.shape, x.dtype),
        mesh=mesh,
        scratch_types=[
            pltpu.VMEM((32,), jnp.int32),
            pltpu.VMEM((_CH, x.shape[-1]), jnp.float32),
            pltpu.VMEM((_CH, x.shape[-1]), jnp.float32),
            pltpu.SemaphoreType.DMA,
        ],
    )
    return f(x, keep_k.astype(jnp.int32), mask_token)


# speedup vs baseline: 12.4353x; 1.4353x over previous
import jax
import jax.numpy as jnp
from jax import lax
from jax.experimental import pallas as pl
from jax.experimental.pallas import tpu as pltpu
from jax.experimental.pallas import tpu_sc as plsc

_CH = 32
_NCH = 1024 // _CH
_NW = 32


def _sc_body(x_hbm, kk_hbm, tok_hbm, out_hbm,
             kk_v, mask_buf, buf_a, sem_fill):
    c = lax.axis_index("c")
    s = lax.axis_index("s")
    w = s * 2 + c

    pltpu.sync_copy(kk_hbm, kk_v.at[pl.ds(0, 16)])

    for i in range(_CH):
        pltpu.async_copy(tok_hbm, mask_buf.at[i], sem_fill)
    pltpu.make_async_copy(x_hbm.at[0, 0, pl.ds(0, _CH)], mask_buf,
                          sem_fill).wait()

    def task(j, carry):
        b = j // 16
        t = j % 16
        kk = kk_v[pl.ds(t, 16)][0]
        ci = lax.rem(w + j, _NW)
        lo = ci * _CH
        hi = lo + _CH
        sl = pl.ds(lo, _CH)

        is_copy = kk >= hi
        is_fill = kk <= lo
        is_mixed = jnp.logical_and(kk > lo, kk < hi)

        @pl.when(is_fill)
        def _fill():
            pltpu.sync_copy(mask_buf, out_hbm.at[b, t, sl])

        @pl.when(is_copy)
        def _copy():
            pltpu.sync_copy(x_hbm.at[b, t, sl], buf_a)
            pltpu.sync_copy(buf_a, out_hbm.at[b, t, sl])

        @pl.when(is_mixed)
        def _mixed():
            pltpu.sync_copy(x_hbm.at[b, t, sl], buf_a)
            r = kk - lo

            def row(n, rc):
                @pl.when(n < r)
                def _():
                    pltpu.sync_copy(buf_a.at[n], out_hbm.at[b, t, lo + n])

                @pl.when(n >= r)
                def _():
                    pltpu.sync_copy(mask_buf.at[0], out_hbm.at[b, t, lo + n])

                return rc

            lax.fori_loop(0, _CH, row, 0)

        return carry

    lax.fori_loop(0, _NW, task, 0)


def kernel(x, keep_k, mask_token):
    D = x.shape[-1]
    mesh = plsc.VectorSubcoreMesh(core_axis_name="c", subcore_axis_name="s",
                                  num_cores=2, num_subcores=16)
    f = pl.kernel(
        _sc_body,
        out_type=jax.ShapeDtypeStruct(x.shape, x.dtype),
        mesh=mesh,
        scratch_types=[
            pltpu.VMEM((32,), jnp.int32),
            pltpu.VMEM((_CH, D), jnp.float32),
            pltpu.VMEM((_CH, D), jnp.float32),
            pltpu.SemaphoreType.DMA,
        ],
    )
    return f(x, keep_k.astype(jnp.int32), mask_token)


# speedup vs baseline: 13.1129x; 1.0545x over previous
import jax
import jax.numpy as jnp
from jax import lax
from jax.experimental import pallas as pl
from jax.experimental.pallas import tpu as pltpu
from jax.experimental.pallas import tpu_sc as plsc

_CH = 32
_NCH = 1024 // _CH
_NW = 32


def _sc_body(x_hbm, kk_hbm, tok_hbm, out_hbm,
             kk_v, mask_buf, buf_a, buf_m, sem_fill, sem_mix):
    c = lax.axis_index("c")
    s = lax.axis_index("s")
    w = s * 2 + c

    pltpu.sync_copy(kk_hbm, kk_v.at[pl.ds(0, 16)])

    for i in range(_CH):
        pltpu.async_copy(tok_hbm, mask_buf.at[i], sem_fill)
    pltpu.make_async_copy(x_hbm.at[0, 0, pl.ds(0, _CH)], mask_buf,
                          sem_fill).wait()

    def task(j, carry):
        b = j // 16
        t = j % 16
        kk = kk_v[pl.ds(t, 16)][0]
        ci = lax.rem(w + j, _NW)
        lo = ci * _CH
        hi = lo + _CH
        sl = pl.ds(lo, _CH)

        is_copy = kk >= hi
        is_fill = kk <= lo
        is_mixed = jnp.logical_and(kk > lo, kk < hi)

        @pl.when(is_fill)
        def _fill():
            pltpu.async_copy(mask_buf, out_hbm.at[b, t, sl], sem_fill)

        @pl.when(is_copy)
        def _copy():
            pltpu.sync_copy(x_hbm.at[b, t, sl], buf_a)
            pltpu.sync_copy(buf_a, out_hbm.at[b, t, sl])

        @pl.when(is_mixed)
        def _mixed():
            m_used = carry[1]

            @pl.when(m_used > 0)
            def _():
                pltpu.make_async_copy(x_hbm.at[0, 0, pl.ds(0, _CH)],
                                      out_hbm.at[0, 0, pl.ds(0, _CH)],
                                      sem_mix).wait()

            pltpu.sync_copy(x_hbm.at[b, t, sl], buf_m)
            r = kk - lo

            def row(n, rc):
                @pl.when(n < r)
                def _():
                    pltpu.async_copy(buf_m.at[n], out_hbm.at[b, t, lo + n],
                                     sem_mix)

                @pl.when(n >= r)
                def _():
                    pltpu.async_copy(mask_buf.at[0], out_hbm.at[b, t, lo + n],
                                     sem_mix)

                return rc

            lax.fori_loop(0, _CH, row, 0)

        one = jnp.int32(1)
        zero = jnp.int32(0)
        nfill = carry[0] + jnp.where(is_fill, one, zero)
        m_used = carry[1] + jnp.where(is_mixed, one, zero)
        return nfill, m_used

    nfill, m_used = lax.fori_loop(0, _NW, task, (jnp.int32(0), jnp.int32(0)))

    def drain_fill(i, carry):
        pltpu.make_async_copy(x_hbm.at[0, 0, pl.ds(0, _CH)],
                              out_hbm.at[0, 0, pl.ds(0, _CH)], sem_fill).wait()
        return carry

    lax.fori_loop(0, nfill, drain_fill, 0)

    @pl.when(m_used > 0)
    def _():
        pltpu.make_async_copy(x_hbm.at[0, 0, pl.ds(0, _CH)],
                              out_hbm.at[0, 0, pl.ds(0, _CH)], sem_mix).wait()


def kernel(x, keep_k, mask_token):
    D = x.shape[-1]
    mesh = plsc.VectorSubcoreMesh(core_axis_name="c", subcore_axis_name="s",
                                  num_cores=2, num_subcores=16)
    f = pl.kernel(
        _sc_body,
        out_type=jax.ShapeDtypeStruct(x.shape, x.dtype),
        mesh=mesh,
        scratch_types=[
            pltpu.VMEM((32,), jnp.int32),
            pltpu.VMEM((_CH, D), jnp.float32),
            pltpu.VMEM((_CH, D), jnp.float32),
            pltpu.VMEM((_CH, D), jnp.float32),
            pltpu.SemaphoreType.DMA,
            pltpu.SemaphoreType.DMA,
        ],
    )
    return f(x, keep_k.astype(jnp.int32), mask_token)


# speedup vs baseline: 14.2897x; 1.0897x over previous
import jax
import jax.numpy as jnp
from jax import lax
from jax.experimental import pallas as pl
from jax.experimental.pallas import tpu as pltpu
from jax.experimental.pallas import tpu_sc as plsc

_CH = 32
_NCH = 1024 // _CH
_NW = 32


def _sc_body(x_hbm, kk_hbm, tok_hbm, out_hbm,
             kk_v, mask_buf, buf_a, buf_b, buf_m,
             sem_fill, sem_mix, sem_out_a, sem_out_b):
    c = lax.axis_index("c")
    s = lax.axis_index("s")
    w = s * 2 + c

    pltpu.sync_copy(kk_hbm, kk_v.at[pl.ds(0, 16)])

    for i in range(_CH):
        pltpu.async_copy(tok_hbm, mask_buf.at[i], sem_fill)
    pltpu.make_async_copy(x_hbm.at[0, 0, pl.ds(0, _CH)], mask_buf,
                          sem_fill).wait()

    def task(j, carry):
        b = j // 16
        t = j % 16
        kk = kk_v[pl.ds(t, 16)][0]
        ci = lax.rem(w + j, _NW)
        lo = ci * _CH
        hi = lo + _CH
        sl = pl.ds(lo, _CH)

        is_copy = kk >= hi
        is_fill = kk <= lo
        is_mixed = jnp.logical_and(kk > lo, kk < hi)

        @pl.when(is_fill)
        def _fill():
            pltpu.async_copy(mask_buf, out_hbm.at[b, t, sl], sem_fill)

        cnt = carry[2]

        @pl.when(jnp.logical_and(is_copy, lax.rem(cnt, 2) == 0))
        def _copy_a():
            @pl.when(cnt >= 2)
            def _():
                pltpu.make_async_copy(buf_a,
                                      out_hbm.at[0, 0, pl.ds(0, _CH)],
                                      sem_out_a).wait()
            pltpu.sync_copy(x_hbm.at[b, t, sl], buf_a)
            pltpu.async_copy(buf_a, out_hbm.at[b, t, sl], sem_out_a)

        @pl.when(jnp.logical_and(is_copy, lax.rem(cnt, 2) == 1))
        def _copy_b():
            @pl.when(cnt >= 2)
            def _():
                pltpu.make_async_copy(buf_b,
                                      out_hbm.at[0, 0, pl.ds(0, _CH)],
                                      sem_out_b).wait()
            pltpu.sync_copy(x_hbm.at[b, t, sl], buf_b)
            pltpu.async_copy(buf_b, out_hbm.at[b, t, sl], sem_out_b)

        @pl.when(is_mixed)
        def _mixed():
            m_used = carry[1]

            @pl.when(m_used > 0)
            def _():
                pltpu.make_async_copy(buf_m,
                                      out_hbm.at[0, 0, pl.ds(0, _CH)],
                                      sem_mix).wait()

            pltpu.sync_copy(x_hbm.at[b, t, sl], buf_m)
            r = kk - lo

            def row(n, rc):
                @pl.when(n < r)
                def _():
                    pltpu.async_copy(buf_m.at[n], out_hbm.at[b, t, lo + n],
                                     sem_mix)

                @pl.when(n >= r)
                def _():
                    pltpu.async_copy(mask_buf.at[0], out_hbm.at[b, t, lo + n],
                                     sem_mix)

                return rc

            lax.fori_loop(0, _CH, row, 0)

        one = jnp.int32(1)
        zero = jnp.int32(0)
        nfill = carry[0] + jnp.where(is_fill, one, zero)
        m_used = carry[1] + jnp.where(is_mixed, one, zero)
        cnt = cnt + jnp.where(is_copy, one, zero)
        return nfill, m_used, cnt

    nfill, m_used, cnt = lax.fori_loop(
        0, _NW, task, (jnp.int32(0), jnp.int32(0), jnp.int32(0)))

    def drain_fill(i, carry):
        pltpu.make_async_copy(mask_buf,
                              out_hbm.at[0, 0, pl.ds(0, _CH)], sem_fill).wait()
        return carry

    lax.fori_loop(0, nfill, drain_fill, 0)

    @pl.when(m_used > 0)
    def _():
        pltpu.make_async_copy(buf_m,
                              out_hbm.at[0, 0, pl.ds(0, _CH)], sem_mix).wait()

    @pl.when(cnt >= 1)
    def _():
        @pl.when(lax.rem(cnt, 2) == 1)
        def _():
            pltpu.make_async_copy(buf_a,
                                  out_hbm.at[0, 0, pl.ds(0, _CH)],
                                  sem_out_a).wait()

        @pl.when(lax.rem(cnt, 2) == 0)
        def _():
            pltpu.make_async_copy(buf_b,
                                  out_hbm.at[0, 0, pl.ds(0, _CH)],
                                  sem_out_b).wait()

    @pl.when(cnt >= 2)
    def _():
        @pl.when(lax.rem(cnt, 2) == 1)
        def _():
            pltpu.make_async_copy(buf_b,
                                  out_hbm.at[0, 0, pl.ds(0, _CH)],
                                  sem_out_b).wait()

        @pl.when(lax.rem(cnt, 2) == 0)
        def _():
            pltpu.make_async_copy(buf_a,
                                  out_hbm.at[0, 0, pl.ds(0, _CH)],
                                  sem_out_a).wait()


def kernel(x, keep_k, mask_token):
    D = x.shape[-1]
    mesh = plsc.VectorSubcoreMesh(core_axis_name="c", subcore_axis_name="s",
                                  num_cores=2, num_subcores=16)
    f = pl.kernel(
        _sc_body,
        out_type=jax.ShapeDtypeStruct(x.shape, x.dtype),
        mesh=mesh,
        scratch_types=[
            pltpu.VMEM((32,), jnp.int32),
            pltpu.VMEM((_CH, D), jnp.float32),
            pltpu.VMEM((_CH, D), jnp.float32),
            pltpu.VMEM((_CH, D), jnp.float32),
            pltpu.VMEM((_CH, D), jnp.float32),
            pltpu.SemaphoreType.DMA,
            pltpu.SemaphoreType.DMA,
            pltpu.SemaphoreType.DMA,
            pltpu.SemaphoreType.DMA,
        ],
    )
    return f(x, keep_k.astype(jnp.int32), mask_token)


# speedup vs baseline: 14.7002x; 1.0287x over previous
import jax
import jax.numpy as jnp
from jax import lax
from jax.experimental import pallas as pl
from jax.experimental.pallas import tpu as pltpu
from jax.experimental.pallas import tpu_sc as plsc

_CH = 32
_NCH = 1024 // _CH
_NW = 32


def _sc_body(x_hbm, kk_hbm, tok_hbm, out_hbm,
             kk_v, mask_buf, buf_a, buf_b, buf_m,
             sem_fill, sem_mix, sem_out_a, sem_out_b, sem_in_a, sem_in_b):
    c = lax.axis_index("c")
    s = lax.axis_index("s")
    w = s * 2 + c

    pltpu.sync_copy(kk_hbm, kk_v.at[pl.ds(0, 16)])

    for i in range(_CH):
        pltpu.async_copy(tok_hbm, mask_buf.at[i], sem_fill)
    pltpu.make_async_copy(x_hbm.at[0, 0, pl.ds(0, _CH)], mask_buf,
                          sem_fill).wait()

    def task(j, carry):
        b = j // 16
        t = j % 16
        kk = kk_v[pl.ds(t, 16)][0]
        ci = lax.rem(w + j, _NW)
        lo = ci * _CH
        hi = lo + _CH
        sl = pl.ds(lo, _CH)

        is_copy = kk >= hi
        is_fill = kk <= lo
        is_mixed = jnp.logical_and(kk > lo, kk < hi)

        @pl.when(is_fill)
        def _fill():
            pltpu.async_copy(mask_buf, out_hbm.at[b, t, sl], sem_fill)

        cnt = carry[2]
        pj = carry[3]
        plo = pl.multiple_of(carry[4], _CH)

        @pl.when(jnp.logical_and(is_copy, lax.rem(cnt, 2) == 0))
        def _copy_a():
            @pl.when(cnt >= 2)
            def _():
                pltpu.make_async_copy(buf_a,
                                      out_hbm.at[0, 0, pl.ds(0, _CH)],
                                      sem_out_a).wait()
            pltpu.async_copy(x_hbm.at[b, t, sl], buf_a, sem_in_a)

            @pl.when(cnt >= 1)
            def _():
                pltpu.make_async_copy(x_hbm.at[0, 0, pl.ds(0, _CH)],
                                      buf_b, sem_in_b).wait()
                pltpu.async_copy(buf_b,
                                 out_hbm.at[pj // 16, pj % 16,
                                            pl.ds(plo, _CH)], sem_out_b)

        @pl.when(jnp.logical_and(is_copy, lax.rem(cnt, 2) == 1))
        def _copy_b():
            @pl.when(cnt >= 2)
            def _():
                pltpu.make_async_copy(buf_b,
                                      out_hbm.at[0, 0, pl.ds(0, _CH)],
                                      sem_out_b).wait()
            pltpu.async_copy(x_hbm.at[b, t, sl], buf_b, sem_in_b)

            pltpu.make_async_copy(x_hbm.at[0, 0, pl.ds(0, _CH)],
                                  buf_a, sem_in_a).wait()
            pltpu.async_copy(buf_a,
                             out_hbm.at[pj // 16, pj % 16,
                                        pl.ds(plo, _CH)], sem_out_a)

        @pl.when(is_mixed)
        def _mixed():
            m_used = carry[1]

            @pl.when(m_used > 0)
            def _():
                pltpu.make_async_copy(buf_m,
                                      out_hbm.at[0, 0, pl.ds(0, _CH)],
                                      sem_mix).wait()

            pltpu.sync_copy(x_hbm.at[b, t, sl], buf_m)
            r = kk - lo

            def row(n, rc):
                @pl.when(n < r)
                def _():
                    pltpu.async_copy(buf_m.at[n], out_hbm.at[b, t, lo + n],
                                     sem_mix)

                @pl.when(n >= r)
                def _():
                    pltpu.async_copy(mask_buf.at[0], out_hbm.at[b, t, lo + n],
                                     sem_mix)

                return rc

            lax.fori_loop(0, _CH, row, 0)

        one = jnp.int32(1)
        zero = jnp.int32(0)
        nfill = carry[0] + jnp.where(is_fill, one, zero)
        m_used = carry[1] + jnp.where(is_mixed, one, zero)
        pj = jnp.where(is_copy, j, pj).astype(pj.dtype)
        plo = jnp.where(is_copy, lo, plo).astype(plo.dtype)
        cnt = cnt + jnp.where(is_copy, one, zero)
        return nfill, m_used, cnt, pj, plo

    nfill, m_used, cnt, pj, plo = lax.fori_loop(
        0, _NW, task,
        (jnp.int32(0), jnp.int32(0), jnp.int32(0), jnp.int32(0), jnp.int32(0)))

    plo = pl.multiple_of(plo, _CH)

    @pl.when(cnt >= 1)
    def _():
        @pl.when(lax.rem(cnt, 2) == 1)
        def _():
            pltpu.make_async_copy(x_hbm.at[0, 0, pl.ds(0, _CH)],
                                  buf_a, sem_in_a).wait()
            pltpu.async_copy(buf_a,
                             out_hbm.at[pj // 16, pj % 16, pl.ds(plo, _CH)],
                             sem_out_a)

        @pl.when(lax.rem(cnt, 2) == 0)
        def _():
            pltpu.make_async_copy(x_hbm.at[0, 0, pl.ds(0, _CH)],
                                  buf_b, sem_in_b).wait()
            pltpu.async_copy(buf_b,
                             out_hbm.at[pj // 16, pj % 16, pl.ds(plo, _CH)],
                             sem_out_b)

    def drain_fill(i, carry):
        pltpu.make_async_copy(mask_buf,
                              out_hbm.at[0, 0, pl.ds(0, _CH)], sem_fill).wait()
        return carry

    lax.fori_loop(0, nfill, drain_fill, 0)

    @pl.when(m_used > 0)
    def _():
        pltpu.make_async_copy(buf_m,
                              out_hbm.at[0, 0, pl.ds(0, _CH)], sem_mix).wait()

    @pl.when(cnt >= 1)
    def _():
        @pl.when(lax.rem(cnt, 2) == 1)
        def _():
            pltpu.make_async_copy(buf_a,
                                  out_hbm.at[0, 0, pl.ds(0, _CH)],
                                  sem_out_a).wait()

        @pl.when(lax.rem(cnt, 2) == 0)
        def _():
            pltpu.make_async_copy(buf_b,
                                  out_hbm.at[0, 0, pl.ds(0, _CH)],
                                  sem_out_b).wait()

    @pl.when(cnt >= 2)
    def _():
        @pl.when(lax.rem(cnt, 2) == 1)
        def _():
            pltpu.make_async_copy(buf_b,
                                  out_hbm.at[0, 0, pl.ds(0, _CH)],
                                  sem_out_b).wait()

        @pl.when(lax.rem(cnt, 2) == 0)
        def _():
            pltpu.make_async_copy(buf_a,
                                  out_hbm.at[0, 0, pl.ds(0, _CH)],
                                  sem_out_a).wait()


def kernel(x, keep_k, mask_token):
    D = x.shape[-1]
    mesh = plsc.VectorSubcoreMesh(core_axis_name="c", subcore_axis_name="s",
                                  num_cores=2, num_subcores=16)
    f = pl.kernel(
        _sc_body,
        out_type=jax.ShapeDtypeStruct(x.shape, x.dtype),
        mesh=mesh,
        scratch_types=[
            pltpu.VMEM((32,), jnp.int32),
            pltpu.VMEM((_CH, D), jnp.float32),
            pltpu.VMEM((_CH, D), jnp.float32),
            pltpu.VMEM((_CH, D), jnp.float32),
            pltpu.VMEM((_CH, D), jnp.float32),
            pltpu.SemaphoreType.DMA,
            pltpu.SemaphoreType.DMA,
            pltpu.SemaphoreType.DMA,
            pltpu.SemaphoreType.DMA,
            pltpu.SemaphoreType.DMA,
            pltpu.SemaphoreType.DMA,
        ],
    )
    return f(x, keep_k.astype(jnp.int32), mask_token)
